# Initial kernel scaffold; baseline (speedup 1.0000x reference)
import jax
import jax.numpy as jnp
from jax.experimental import pallas as pl


def kernel(items, query_words, word_table, item_table, W_q, b_q):
    raise NotImplementedError("write your pallas kernel here")


# speedup vs baseline: 1.6756x; 1.6756x over previous
import functools

import jax
import jax.numpy as jnp
import numpy as np
from jax import lax
from jax.experimental import pallas as pl
from jax.experimental.pallas import tpu as pltpu
from jax.experimental.pallas import tpu_sc as plsc

B = 16384
L = 50
D = 32

NC = 2
NS = 16
NW = NC * NS
BPW = B // NW

QMIN = 100
CH = 32
NCHUNK = BPW // CH
GPC = CH * L // QMIN
QRPW = BPW * L // QMIN

IMIN = 64
NIT = BPW // IMIN

_mesh = plsc.VectorSubcoreMesh(
    core_axis_name="c", subcore_axis_name="s", num_cores=NC, num_subcores=NS
)


@functools.partial(
    pl.kernel,
    out_type=(
        jax.ShapeDtypeStruct((B, D), jnp.float32),
        jax.ShapeDtypeStruct((B, D), jnp.float32),
    ),
    mesh=_mesh,
    compiler_params=pltpu.CompilerParams(use_tc_tiling_on_sc=False),
    scratch_types=[
        pltpu.VMEM((GPC, QMIN), jnp.int32),
        pltpu.VMEM((CH * L, D), jnp.float32),
        pltpu.VMEM((CH, D), jnp.float32),
        pltpu.VMEM((NIT, IMIN), jnp.int32),
        pltpu.VMEM((IMIN, D), jnp.float32),
        pltpu.SemaphoreType.DMA,
    ],
)
def _sc_gather_pool(
    qw2_hbm, items2_hbm, word_hbm, item_hbm,
    pooled_hbm, iout_hbm,
    idx_v, rows_v, pooled_v, iidx_v, irows_v, sem,
):
    wid = lax.axis_index("s") * NC + lax.axis_index("c")
    base = wid * BPW

    pltpu.sync_copy(items2_hbm.at[pl.ds(wid * NIT, NIT)], iidx_v)

    def chunk_body(c, carry):
        row0 = wid * QRPW + c * GPC
        pltpu.sync_copy(qw2_hbm.at[pl.ds(row0, GPC)], idx_v)
        cps = [
            pltpu.async_copy(
                word_hbm.at[idx_v.at[j]],
                rows_v.at[pl.ds(j * QMIN, QMIN)],
                sem,
            )
            for j in range(GPC)
        ]
        for cp in cps:
            cp.wait()

        def item_body(b, acc_carry):
            r0 = b * L
            acc0 = rows_v[r0, pl.ds(0, 16)]
            acc1 = rows_v[r0, pl.ds(16, 16)]
            for l in range(1, L):
                acc0 = acc0 + rows_v[r0 + l, pl.ds(0, 16)]
                acc1 = acc1 + rows_v[r0 + l, pl.ds(16, 16)]
            pooled_v[b, pl.ds(0, 16)] = acc0
            pooled_v[b, pl.ds(16, 16)] = acc1
            return acc_carry

        lax.fori_loop(0, CH, item_body, 0)
        pltpu.sync_copy(pooled_v, pooled_hbm.at[pl.ds(base + c * CH, CH)])
        return carry

    lax.fori_loop(0, NCHUNK, chunk_body, 0)

    for t in range(NIT):
        pltpu.async_copy(item_hbm.at[iidx_v.at[t]], irows_v, sem).wait()
        pltpu.sync_copy(irows_v, iout_hbm.at[pl.ds(base + t * IMIN, IMIN)])


_TB = 2048


def _tc_proj_body(x_ref, w_ref, b_ref, o_ref):
    x = x_ref[...] * np.float32(1.0 / L)
    y = lax.dot_general(
        x, w_ref[...], (((1,), (1,)), ((), ())),
        preferred_element_type=jnp.float32,
    )
    o_ref[...] = jnp.tanh(y + b_ref[...])


_tc_proj = pl.pallas_call(
    _tc_proj_body,
    out_shape=jax.ShapeDtypeStruct((B, D), jnp.float32),
    grid=(B // _TB,),
    in_specs=[
        pl.BlockSpec((_TB, D), lambda i: (i, 0)),
        pl.BlockSpec((D, D), lambda i: (0, 0)),
        pl.BlockSpec((1, D), lambda i: (0, 0)),
    ],
    out_specs=pl.BlockSpec((_TB, D), lambda i: (i, 0)),
)


def kernel(items, query_words, word_table, item_table, W_q, b_q):
    qw2 = query_words.astype(jnp.int32).reshape(B * L // QMIN, QMIN)
    items2 = items.astype(jnp.int32).reshape(B // IMIN, IMIN)
    pooled_sum, item_emb = _sc_gather_pool(qw2, items2, word_table, item_table)
    q = _tc_proj(pooled_sum, W_q, b_q.reshape(1, D))
    return (q, item_emb)


# speedup vs baseline: 1.6792x; 1.0021x over previous
import functools

import jax
import jax.numpy as jnp
import numpy as np
from jax import lax
from jax.experimental import pallas as pl
from jax.experimental.pallas import tpu as pltpu
from jax.experimental.pallas import tpu_sc as plsc

B = 16384
L = 50
D = 32

NC = 2
NS = 16
NW = NC * NS
BPW = B // NW

CH = 32
NCHUNK = BPW // CH

ITCH = 128
NIT = BPW // ITCH

_mesh = plsc.VectorSubcoreMesh(
    core_axis_name="c", subcore_axis_name="s", num_cores=NC, num_subcores=NS
)


@functools.partial(
    pl.kernel,
    out_type=(
        jax.ShapeDtypeStruct((B, D), jnp.float32),
        jax.ShapeDtypeStruct((B, D), jnp.float32),
    ),
    mesh=_mesh,
    compiler_params=pltpu.CompilerParams(use_tc_tiling_on_sc=False),
    scratch_types=[
        pltpu.VMEM((CH, L), jnp.int32),
        pltpu.VMEM((CH, L, D), jnp.float32),
        pltpu.VMEM((CH, D), jnp.float32),
        pltpu.VMEM((BPW,), jnp.int32),
        pltpu.VMEM((ITCH, D), jnp.float32),
        pltpu.SemaphoreType.DMA,
    ],
)
def _sc_gather_pool(
    items_hbm, qw_hbm, word_hbm, item_hbm,
    pooled_hbm, iout_hbm,
    idx_v, rows_v, pooled_v, iidx_v, irows_v, sem,
):
    wid = lax.axis_index("s") * NC + lax.axis_index("c")
    base = wid * BPW

    pltpu.sync_copy(items_hbm.at[pl.ds(base, BPW)], iidx_v)

    def chunk_body(c, carry):
        pltpu.sync_copy(qw_hbm.at[pl.ds(base + c * CH, CH)], idx_v)
        cps = [
            pltpu.async_copy(word_hbm.at[idx_v.at[b]], rows_v.at[b], sem)
            for b in range(CH)
        ]
        for cp in cps:
            cp.wait()

        def item_body(b, acc_carry):
            acc0 = rows_v[b, 0, pl.ds(0, 16)]
            acc1 = rows_v[b, 0, pl.ds(16, 16)]
            for l in range(1, L):
                acc0 = acc0 + rows_v[b, l, pl.ds(0, 16)]
                acc1 = acc1 + rows_v[b, l, pl.ds(16, 16)]
            pooled_v[b, pl.ds(0, 16)] = acc0
            pooled_v[b, pl.ds(16, 16)] = acc1
            return acc_carry

        lax.fori_loop(0, CH, item_body, 0)
        pltpu.sync_copy(pooled_v, pooled_hbm.at[pl.ds(base + c * CH, CH)])
        return carry

    lax.fori_loop(0, NCHUNK, chunk_body, 0)

    for t in range(NIT):
        pltpu.async_copy(
            item_hbm.at[iidx_v.at[pl.ds(t * ITCH, ITCH)]], irows_v, sem
        ).wait()
        pltpu.sync_copy(irows_v, iout_hbm.at[pl.ds(base + t * ITCH, ITCH)])


_TB = 2048


def _tc_proj_body(x_ref, w_ref, b_ref, o_ref):
    x = x_ref[...] * np.float32(1.0 / L)
    y = lax.dot_general(
        x, w_ref[...], (((1,), (1,)), ((), ())),
        preferred_element_type=jnp.float32,
    )
    o_ref[...] = jnp.tanh(y + b_ref[...])


_tc_proj = pl.pallas_call(
    _tc_proj_body,
    out_shape=jax.ShapeDtypeStruct((B, D), jnp.float32),
    grid=(B // _TB,),
    in_specs=[
        pl.BlockSpec((_TB, D), lambda i: (i, 0)),
        pl.BlockSpec((D, D), lambda i: (0, 0)),
        pl.BlockSpec((1, D), lambda i: (0, 0)),
    ],
    out_specs=pl.BlockSpec((_TB, D), lambda i: (i, 0)),
)


def kernel(items, query_words, word_table, item_table, W_q, b_q):
    items = items.astype(jnp.int32)
    query_words = query_words.astype(jnp.int32)
    pooled_sum, item_emb = _sc_gather_pool(
        items, query_words, word_table, item_table
    )
    q = _tc_proj(pooled_sum, W_q, b_q.reshape(1, D))
    return (q, item_emb)


# speedup vs baseline: 1.7002x; 1.0125x over previous
import functools

import jax
import jax.numpy as jnp
import numpy as np
from jax import lax
from jax.experimental import pallas as pl
from jax.experimental.pallas import tpu as pltpu
from jax.experimental.pallas import tpu_sc as plsc

B = 16384
L = 50
D = 32

NC = 2
NS = 16
NW = NC * NS
BPW = B // NW

FCH = 32
NFCH = BPW // FCH

CH = 64
NCHUNK = BPW // CH
GPC = CH * L // 128

ITCH = 128
NIT = BPW // ITCH

_mesh = plsc.VectorSubcoreMesh(
    core_axis_name="c", subcore_axis_name="s", num_cores=NC, num_subcores=NS
)


@functools.partial(
    pl.kernel,
    out_type=jax.ShapeDtypeStruct((B * L,), jnp.int32),
    mesh=_mesh,
    compiler_params=pltpu.CompilerParams(use_tc_tiling_on_sc=True),
    scratch_types=[
        pltpu.VMEM((FCH, L), jnp.int32),
        pltpu.VMEM((FCH * L,), jnp.int32),
        pltpu.SemaphoreType.DMA,
    ],
)
def _sc_flatten_idx(qw_hbm, out_hbm, v2, v1, sem):
    wid = lax.axis_index("s") * NC + lax.axis_index("c")
    base = wid * BPW

    def chunk(c, carry):
        pltpu.sync_copy(qw_hbm.at[pl.ds(base + c * FCH, FCH)], v2)

        def row(r, rc):
            o = r * L
            v1[pl.ds(o, 16)] = v2[r, pl.ds(0, 16)]
            v1[pl.ds(o + 16, 16)] = v2[r, pl.ds(16, 16)]
            v1[pl.ds(o + 32, 16)] = v2[r, pl.ds(32, 16)]
            v1[pl.ds(o + 34, 16)] = v2[r, pl.ds(34, 16)]
            return rc

        lax.fori_loop(0, FCH, row, 0)
        pltpu.sync_copy(v1, out_hbm.at[pl.ds((base + c * FCH) * L, FCH * L)])
        return carry

    lax.fori_loop(0, NFCH, chunk, 0)


@functools.partial(
    pl.kernel,
    out_type=(
        jax.ShapeDtypeStruct((B, D), jnp.float32),
        jax.ShapeDtypeStruct((B, D), jnp.float32),
    ),
    mesh=_mesh,
    compiler_params=pltpu.CompilerParams(use_tc_tiling_on_sc=False),
    scratch_types=[
        pltpu.VMEM((CH * L,), jnp.int32),
        pltpu.VMEM((CH * L, D), jnp.float32),
        pltpu.VMEM((CH, D), jnp.float32),
        pltpu.VMEM((BPW,), jnp.int32),
        pltpu.VMEM((ITCH, D), jnp.float32),
        pltpu.SemaphoreType.DMA,
    ],
)
def _sc_gather_pool(
    items_hbm, qwf_hbm, word_hbm, item_hbm,
    pooled_hbm, iout_hbm,
    idx_v, rows_v, pooled_v, iidx_v, irows_v, sem,
):
    wid = lax.axis_index("s") * NC + lax.axis_index("c")
    base = wid * BPW

    pltpu.sync_copy(items_hbm.at[pl.ds(base, BPW)], iidx_v)

    def chunk_body(c, carry):
        pltpu.sync_copy(
            qwf_hbm.at[pl.ds((base + c * CH) * L, CH * L)], idx_v
        )
        cps = [
            pltpu.async_copy(
                word_hbm.at[idx_v.at[pl.ds(j * 128, 128)]],
                rows_v.at[pl.ds(j * 128, 128)],
                sem,
            )
            for j in range(GPC)
        ]
        for cp in cps:
            cp.wait()

        def item_body(b, acc_carry):
            r0 = b * L
            acc0 = rows_v[r0, pl.ds(0, 16)]
            acc1 = rows_v[r0, pl.ds(16, 16)]
            for l in range(1, L):
                acc0 = acc0 + rows_v[r0 + l, pl.ds(0, 16)]
                acc1 = acc1 + rows_v[r0 + l, pl.ds(16, 16)]
            pooled_v[b, pl.ds(0, 16)] = acc0
            pooled_v[b, pl.ds(16, 16)] = acc1
            return acc_carry

        lax.fori_loop(0, CH, item_body, 0)
        pltpu.sync_copy(pooled_v, pooled_hbm.at[pl.ds(base + c * CH, CH)])
        return carry

    lax.fori_loop(0, NCHUNK, chunk_body, 0)

    for t in range(NIT):
        pltpu.async_copy(
            item_hbm.at[iidx_v.at[pl.ds(t * ITCH, ITCH)]], irows_v, sem
        ).wait()
        pltpu.sync_copy(irows_v, iout_hbm.at[pl.ds(base + t * ITCH, ITCH)])


_TB = 2048


def _tc_proj_body(x_ref, w_ref, b_ref, o_ref):
    x = x_ref[...] * np.float32(1.0 / L)
    y = lax.dot_general(
        x, w_ref[...], (((1,), (1,)), ((), ())),
        preferred_element_type=jnp.float32,
    )
    o_ref[...] = jnp.tanh(y + b_ref[...])


_tc_proj = pl.pallas_call(
    _tc_proj_body,
    out_shape=jax.ShapeDtypeStruct((B, D), jnp.float32),
    grid=(B // _TB,),
    in_specs=[
        pl.BlockSpec((_TB, D), lambda i: (i, 0)),
        pl.BlockSpec((D, D), lambda i: (0, 0)),
        pl.BlockSpec((1, D), lambda i: (0, 0)),
    ],
    out_specs=pl.BlockSpec((_TB, D), lambda i: (i, 0)),
)


def kernel(items, query_words, word_table, item_table, W_q, b_q):
    items = items.astype(jnp.int32)
    query_words = query_words.astype(jnp.int32)
    qw_flat = _sc_flatten_idx(query_words)
    pooled_sum, item_emb = _sc_gather_pool(
        items, qw_flat, word_table, item_table
    )
    q = _tc_proj(pooled_sum, W_q, b_q.reshape(1, D))
    return (q, item_emb)
